# baseline (speedup 1.0000x reference)
import functools
import math

import jax
import jax.numpy as jnp
import numpy as np
from jax import lax
from jax.experimental import pallas as pl
from jax.experimental.pallas import tpu as pltpu
from jax.experimental.pallas import tpu_sc as plsc

VOCAB = 100000
D = 768
B = 4
T = 8192
N_ROWS = B * T
SCALE = math.sqrt(D)
LANES = 16
C = 16


def _pe_table():
    positions = np.arange(T, dtype=np.float32)[:, None]
    i = np.arange(0, D, 2, dtype=np.float32)
    denominator = np.exp(i / D * math.log(10000.0))
    pe = np.zeros((T, D), dtype=np.float32)
    pe[:, 0::2] = np.sin(positions / denominator)
    pe[:, 1::2] = np.cos(positions / denominator)
    return pe


def _packed_pe_table():
    import ml_dtypes
    pe = _pe_table()
    bits = pe.astype(ml_dtypes.bfloat16).view(np.uint16).astype(np.uint32)
    blk = bits.reshape(T, D // 32, 2, 16)
    packed = blk[:, :, 0, :] | (blk[:, :, 1, :] << 16)
    return packed.reshape(T, D // 2).view(np.int32)


_PE_PACKED = _packed_pe_table()


def _make_sc_kernel():
    info = plsc.get_sparse_core_info()
    nc, ns = info.num_cores, info.num_subcores
    nw = nc * ns
    t_per_w = T // nw
    n_tc = t_per_w // C
    mesh = plsc.VectorSubcoreMesh(core_axis_name="c", subcore_axis_name="s")

    @functools.partial(
        pl.kernel,
        mesh=mesh,
        out_type=jax.ShapeDtypeStruct((N_ROWS, D), jnp.float32),
        scratch_types=[
            pltpu.VMEM((B, t_per_w), jnp.int32),
            [[pltpu.VMEM((C, D), jnp.float32)] * 2] * 4,
            [pltpu.VMEM((C, D // 2), jnp.int32)] * 2,
            [[pltpu.SemaphoreType.DMA] * 2] * 4,
            [[pltpu.SemaphoreType.DMA] * 2] * 4,
            [pltpu.SemaphoreType.DMA] * 2,
        ],
    )
    def k(x_hbm, we_hbm, pe_hbm, out_hbm, idx_all, rows, pes, sg, so, sp):
        wid = lax.axis_index("s") * nc + lax.axis_index("c")
        t0 = wid * t_per_w

        def gather(bh, i, tc, g):
            b = 2 * bh + i
            return pltpu.make_async_copy(
                we_hbm.at[idx_all.at[b, pl.ds(tc * C, C)]],
                rows[g][i], sg[g][i])

        def out_copy(bh, i, tc, g):
            b = 2 * bh + i
            return pltpu.make_async_copy(
                rows[g][i], out_hbm.at[pl.ds(b * T + t0 + tc * C, C)],
                so[g][i])

        def pe_copy(tc, par):
            return pltpu.make_async_copy(
                pe_hbm.at[pl.ds(t0 + tc * C, C)], pes[par], sp[par])

        for b in range(B):
            pltpu.sync_copy(x_hbm.at[pl.ds(b * T + t0, t_per_w)],
                            idx_all.at[b])
        pe_copy(0, 0).start()
        for i in range(2):
            gather(0, i, 0, 0).start()
        for i in range(2):
            gather(1, i, 0, 1).start()

        def tco_body(tco, _):
            for p in range(4):
                tc = tco * 2 + p // 2
                bh = p % 2
                par = p // 2
                g2 = (p + 2) % 4
                for i in range(2):
                    gather(bh, i, tc, p).wait()
                if bh == 0:
                    pe_copy(tc, par).wait()

                    @pl.when(tc + 1 < n_tc)
                    def _():
                        pe_copy(tc + 1, 1 - par).start()

                @pl.when(tc > 0)
                def _():
                    for i in range(2):
                        out_copy(bh, i, tc - 1, g2).wait()

                @pl.when(tc + 1 < n_tc)
                def _():
                    for i in range(2):
                        gather(bh, i, tc + 1, g2).start()

                def row_body(r, _):
                    for kk in range(D // 32):
                        w = pes[par][r, pl.ds(kk * LANES, LANES)]
                        sixteen = jnp.full((LANES,), 16, jnp.int32)
                        himask = jnp.full((LANES,), -65536, jnp.int32)
                        pe_lo = lax.bitcast_convert_type(w << sixteen,
                                                         jnp.float32)
                        pe_hi = lax.bitcast_convert_type(w & himask,
                                                         jnp.float32)
                        sl_lo = pl.ds(kk * 32, LANES)
                        sl_hi = pl.ds(kk * 32 + LANES, LANES)
                        for i in range(2):
                            rows[p][i][r, sl_lo] = (
                                rows[p][i][r, sl_lo] * SCALE + pe_lo)
                            rows[p][i][r, sl_hi] = (
                                rows[p][i][r, sl_hi] * SCALE + pe_hi)
                    return 0

                lax.fori_loop(0, C, row_body, 0)
                for i in range(2):
                    out_copy(bh, i, tc, p).start()
            return 0

        lax.fori_loop(0, n_tc // 2, tco_body, 0)
        for bh in range(2):
            for i in range(2):
                out_copy(bh, i, n_tc - 1, 2 + bh).wait()

    return k


_sc_kernel = _make_sc_kernel()


@jax.jit
def kernel(x, We):
    pe = jnp.asarray(_PE_PACKED)
    flat_idx = x.reshape(-1).astype(jnp.int32)
    out = _sc_kernel(flat_idx, We, pe)
    return out.reshape(B, T, D)


# speedup vs baseline: 1.0636x; 1.0636x over previous
import functools
import math

import jax
import jax.numpy as jnp
import numpy as np
from jax import lax
from jax.experimental import pallas as pl
from jax.experimental.pallas import tpu as pltpu
from jax.experimental.pallas import tpu_sc as plsc

VOCAB = 100000
D = 768
B = 4
T = 8192
N_ROWS = B * T
SCALE = math.sqrt(D)
LANES = 16
C = 16


def _pe_table():
    positions = np.arange(T, dtype=np.float32)[:, None]
    i = np.arange(0, D, 2, dtype=np.float32)
    denominator = np.exp(i / D * math.log(10000.0))
    pe = np.zeros((T, D), dtype=np.float32)
    pe[:, 0::2] = np.sin(positions / denominator)
    pe[:, 1::2] = np.cos(positions / denominator)
    return pe


_PE = _pe_table()


def _make_sc_kernel():
    info = plsc.get_sparse_core_info()
    nc, ns = info.num_cores, info.num_subcores
    nw = nc * ns
    t_per_w = T // nw
    n_tc = t_per_w // C
    mesh = plsc.VectorSubcoreMesh(core_axis_name="c", subcore_axis_name="s")

    @functools.partial(
        pl.kernel,
        mesh=mesh,
        out_type=jax.ShapeDtypeStruct((N_ROWS, D), jnp.float32),
        scratch_types=[
            pltpu.VMEM((B, t_per_w), jnp.int32),
            [[pltpu.VMEM((C, D), jnp.float32)] * 2] * 4,
            [pltpu.VMEM((C, D), jnp.float32)] * 2,
            [[pltpu.SemaphoreType.DMA] * 2] * 4,
            [[pltpu.SemaphoreType.DMA] * 2] * 4,
            [pltpu.SemaphoreType.DMA] * 2,
            [pltpu.SemaphoreType.DMA] * B,
        ],
    )
    def k(x_hbm, we_hbm, pe_hbm, out_hbm, idx_all, rows, pes, sg, so, sp,
          si):
        wid = lax.axis_index("s") * nc + lax.axis_index("c")
        t0 = wid * t_per_w

        def gather(bh, i, tc, g):
            b = 2 * bh + i
            return pltpu.make_async_copy(
                we_hbm.at[idx_all.at[b, pl.ds(tc * C, C)]],
                rows[g][i], sg[g][i])

        def out_copy(bh, i, tc, g):
            b = 2 * bh + i
            return pltpu.make_async_copy(
                rows[g][i], out_hbm.at[pl.ds(b * T + t0 + tc * C, C)],
                so[g][i])

        def pe_copy(tc, par):
            return pltpu.make_async_copy(
                pe_hbm.at[pl.ds(t0 + tc * C, C)], pes[par], sp[par])

        idx_copies = [
            pltpu.make_async_copy(x_hbm.at[pl.ds(b * T + t0, t_per_w)],
                                  idx_all.at[b], si[b])
            for b in range(B)
        ]
        for c in idx_copies:
            c.start()
        pe_copy(0, 0).start()
        for c in idx_copies:
            c.wait()
        for i in range(2):
            gather(0, i, 0, 0).start()
        for i in range(2):
            gather(1, i, 0, 1).start()

        def tco_body(tco, _):
            for p in range(4):
                tc = tco * 2 + p // 2
                bh = p % 2
                par = p // 2
                g2 = (p + 2) % 4
                for i in range(2):
                    gather(bh, i, tc, p).wait()
                if bh == 0:
                    pe_copy(tc, par).wait()

                    @pl.when(tc + 1 < n_tc)
                    def _():
                        pe_copy(tc + 1, 1 - par).start()

                @pl.when(tc > 0)
                def _():
                    for i in range(2):
                        out_copy(bh, i, tc - 1, g2).wait()

                @pl.when(tc + 1 < n_tc)
                def _():
                    for i in range(2):
                        gather(bh, i, tc + 1, g2).start()

                def row_body(r, _):
                    for jj in range(D // LANES):
                        sl = pl.ds(jj * LANES, LANES)
                        pe_vec = pes[par][r, sl]
                        for i in range(2):
                            rows[p][i][r, sl] = (
                                rows[p][i][r, sl] * SCALE + pe_vec)
                    return 0

                lax.fori_loop(0, C, row_body, 0)
                for i in range(2):
                    out_copy(bh, i, tc, p).start()
            return 0

        lax.fori_loop(0, n_tc // 2, tco_body, 0)
        for bh in range(2):
            for i in range(2):
                out_copy(bh, i, n_tc - 1, 2 + bh).wait()

    return k


_sc_kernel = _make_sc_kernel()


@jax.jit
def kernel(x, We):
    pe = jnp.asarray(_PE)
    flat_idx = x.reshape(-1).astype(jnp.int32)
    out = _sc_kernel(flat_idx, We, pe)
    return out.reshape(B, T, D)
